# Initial kernel scaffold; baseline (speedup 1.0000x reference)
import jax
import jax.numpy as jnp
from jax.experimental import pallas as pl


def kernel(x, edge_index_list, triangles_list, node_w, tri_w, res_w, gate_w, gate_b, attn_in_w, attn_in_b, attn_out_w, attn_out_b, lin1_w, lin1_b, lin2_w, lin2_b, n1_w, n1_b, n2_w, n2_b):
    raise NotImplementedError("write your pallas kernel here")


# speedup vs baseline: 2.5812x; 2.5812x over previous
import functools

import jax
import jax.numpy as jnp
from jax import lax
from jax.experimental import pallas as pl
from jax.experimental.pallas import tpu as pltpu
from jax.experimental.pallas import tpu_sc as plsc

N = 50000
D = 128
R = 2
T = 100000
H = 2
DFF = 256

NC = 2
NS = 16
L = 16

T_PAD = 102400
SLAB_A = T_PAD // (NC * NS)
SLAB_B = T_PAD // NS
KB = 128
NB_A = SLAB_A // KB
NB_B = SLAB_B // KB

N_CHUNK = 12800
CH_ROWS = 13056
DUMP = N_CHUNK
G_ROWS = 4 * N_CHUNK

N_PADC = 50176
CNT_FLAT = R * N_PADC
CNT_SLICE = CNT_FLAT // NS
SENT = 1 << 30

BN = 1024
GRID_N = N_PADC // BN

_mesh = plsc.VectorSubcoreMesh(core_axis_name="c", subcore_axis_name="s")


def _zero_fill_1d(ref, n):
    z = jnp.zeros((L,), jnp.float32)

    def body(i, carry):
        ref[pl.ds(i * L, L)] = z
        return carry

    lax.fori_loop(0, n // L, body, 0)


def _gather_body(x_hbm, tri_hbm, m_hbm, cnt_hbm,
                 cnt_sh, idx_a, idx_b, idx_c, gx_a, gx_b, gx_c,
                 ci_a, ci_b, ci_c, cv_a, cv_b, cv_c,
                 rows_a, rows_b, rows_c, m_buf, zc):
    c = lax.axis_index("c")
    s = lax.axis_index("s")
    wid = c * NS + s

    _zero_fill_1d(zc, CNT_SLICE)
    pltpu.sync_copy(zc, cnt_sh.at[pl.ds(s * CNT_SLICE, CNT_SLICE)])
    plsc.subcore_barrier()

    idx_refs = (idx_a, idx_b, idx_c)
    gx_refs = (gx_a, gx_b, gx_c)
    ci_refs = (ci_a, ci_b, ci_c)
    cv_refs = (cv_a, cv_b, cv_c)
    row_refs = (rows_a, rows_b, rows_c)

    for r in range(R):
        def batch(b, carry):
            tb = wid * SLAB_A + b * KB
            for v in range(3):
                pltpu.sync_copy(
                    tri_hbm.at[pl.ds((r * 3 + v) * T_PAD + tb, KB)],
                    idx_refs[v])
            for v in range(3):
                for k in range(KB // L):
                    w = idx_refs[v][pl.ds(k * L, L)]
                    valid = w < N
                    wcl = jnp.minimum(w, N - 1)
                    gx_refs[v][pl.ds(k * L, L)] = wcl
                    ci_refs[v][pl.ds(k * L, L)] = wcl + r * N_PADC
                    cv_refs[v][pl.ds(k * L, L)] = jnp.where(
                        valid, jnp.float32(1.0), jnp.float32(0.0))
            for v in range(3):
                pltpu.sync_copy(x_hbm.at[gx_refs[v]], row_refs[v])
                pltpu.sync_copy(cv_refs[v], cnt_sh.at[ci_refs[v]], add=True)

            def row(j, carry2):
                for k in range(D // L):
                    sl = pl.ds(k * L, L)
                    m_buf[j, sl] = (rows_a[j, sl] + rows_b[j, sl]
                                    + rows_c[j, sl])
                return carry2

            lax.fori_loop(0, KB, row, 0)
            pltpu.sync_copy(m_buf, m_hbm.at[r, pl.ds(tb, KB)])
            return carry

        lax.fori_loop(0, NB_A, batch, 0)

    plsc.subcore_barrier()
    pltpu.sync_copy(cnt_sh.at[pl.ds(s * CNT_SLICE, CNT_SLICE)],
                    cnt_hbm.at[c, pl.ds(s * CNT_SLICE, CNT_SLICE)])


@functools.partial(
    pl.kernel,
    out_type=(
        jax.ShapeDtypeStruct((R, T_PAD, D), jnp.float32),
        jax.ShapeDtypeStruct((NC, CNT_FLAT), jnp.float32),
    ),
    mesh=_mesh,
    scratch_types=[
        pltpu.VMEM_SHARED((CNT_FLAT,), jnp.float32),
        pltpu.VMEM((KB,), jnp.int32), pltpu.VMEM((KB,), jnp.int32),
        pltpu.VMEM((KB,), jnp.int32),
        pltpu.VMEM((KB,), jnp.int32), pltpu.VMEM((KB,), jnp.int32),
        pltpu.VMEM((KB,), jnp.int32),
        pltpu.VMEM((KB,), jnp.int32), pltpu.VMEM((KB,), jnp.int32),
        pltpu.VMEM((KB,), jnp.int32),
        pltpu.VMEM((KB,), jnp.float32), pltpu.VMEM((KB,), jnp.float32),
        pltpu.VMEM((KB,), jnp.float32),
        pltpu.VMEM((KB, D), jnp.float32), pltpu.VMEM((KB, D), jnp.float32),
        pltpu.VMEM((KB, D), jnp.float32), pltpu.VMEM((KB, D), jnp.float32),
        pltpu.VMEM((CNT_SLICE,), jnp.float32),
    ],
)
def _gather_sums(x_hbm, tri_hbm, m_hbm, cnt_hbm, *scratch):
    _gather_body(x_hbm, tri_hbm, m_hbm, cnt_hbm, *scratch)


def _scatter_body(m_hbm, tri_hbm, g_hbm, chunk, idx_v, loc, m_rows):
    c = lax.axis_index("c")
    s = lax.axis_index("s")

    def zrow(j, carry):
        for k in range(D // L):
            m_rows[j, pl.ds(k * L, L)] = jnp.zeros((L,), jnp.float32)
        return carry

    for r in range(R):
        for cid in range(2):
            lo = (2 * c + cid) * N_CHUNK
            lax.fori_loop(0, KB, zrow, 0)
            for q in range(6):
                pltpu.sync_copy(m_rows, chunk.at[pl.ds(s * 816 + q * KB, KB)])
            pltpu.sync_copy(m_rows.at[pl.ds(0, 48)],
                            chunk.at[pl.ds(s * 816 + 6 * KB, 48)])
            plsc.subcore_barrier()

            def batch(b, carry):
                tb = s * SLAB_B + b * KB
                pltpu.sync_copy(m_hbm.at[r, pl.ds(tb, KB)], m_rows)
                for v in range(3):
                    pltpu.sync_copy(
                        tri_hbm.at[pl.ds((r * 3 + v) * T_PAD + tb, KB)],
                        idx_v)
                    for k in range(KB // L):
                        sl = pl.ds(k * L, L)
                        rel = idx_v[sl] - lo
                        ok = (rel >= 0) & (rel < N_CHUNK)
                        loc[sl] = jnp.where(ok, rel, DUMP)
                    pltpu.sync_copy(m_rows, chunk.at[loc], add=True)
                return carry

            lax.fori_loop(0, NB_B, batch, 0)
            plsc.subcore_barrier()
            pltpu.sync_copy(chunk.at[pl.ds(s * 800, 800)],
                            g_hbm.at[r, pl.ds(lo + s * 800, 800)])
            plsc.subcore_barrier()


@functools.partial(
    pl.kernel,
    out_type=jax.ShapeDtypeStruct((R, G_ROWS, D), jnp.float32),
    mesh=_mesh,
    scratch_types=[
        pltpu.VMEM_SHARED((CH_ROWS, D), jnp.float32),
        pltpu.VMEM((KB,), jnp.int32),
        pltpu.VMEM((KB,), jnp.int32),
        pltpu.VMEM((KB, D), jnp.float32),
    ],
)
def _scatter_sums(m_hbm, tri_hbm, g_hbm, *scratch):
    _scatter_body(m_hbm, tri_hbm, g_hbm, *scratch)


def _erf(z):
    a1, a2, a3, a4, a5 = (0.254829592, -0.284496736, 1.421413741,
                          -1.453152027, 1.061405429)
    p = 0.3275911
    az = jnp.abs(z)
    t = 1.0 / (1.0 + p * az)
    poly = ((((a5 * t + a4) * t + a3) * t + a2) * t + a1) * t
    e = 1.0 - poly * jnp.exp(-az * az)
    return jnp.sign(z) * e


def _gelu(z):
    return 0.5 * z * (1.0 + _erf(z * 0.7071067811865476))


def _sigmoid(z):
    return 1.0 / (1.0 + jnp.exp(-z))


def _layernorm(x, w, b):
    mu = jnp.mean(x, axis=-1, keepdims=True)
    var = jnp.mean((x - mu) ** 2, axis=-1, keepdims=True)
    return (x - mu) / jnp.sqrt(var + 1e-5) * w + b


def _dense_body(x_ref, g0_ref, g1_ref, cnt_ref, nwt_ref, twt_ref, rwt_ref,
                gwt_ref, gb_ref, aiwt_ref, aib_ref, aowt_ref, aob_ref,
                l1wt_ref, l1b_ref, l2wt_ref, l2b_ref, n1w_ref, n1b_ref,
                n2w_ref, n2b_ref, out_ref):
    f32 = jnp.float32
    x = x_ref[...]
    cnt4 = cnt_ref[...]
    gb = gb_ref[...]
    aib = aib_ref[...]
    aob = aob_ref[...]
    l1b = l1b_ref[...]
    l2b = l2b_ref[...]
    n1w = n1w_ref[...]; n1b = n1b_ref[...]
    n2w = n2w_ref[...]; n2b = n2b_ref[...]
    gwt = gwt_ref[...]

    dot = functools.partial(jnp.dot, preferred_element_type=f32)

    x_res = dot(x, rwt_ref[...])

    gx = dot(x, gwt[:D, :]) + gb

    lane = lax.broadcasted_iota(jnp.int32, (1, D), 1)
    m0 = (lane < (D // H)).astype(f32)
    m1 = 1.0 - m0
    row = lax.broadcasted_iota(jnp.int32, (D, H), 0)
    col = lax.broadcasted_iota(jnp.int32, (D, H), 1)
    P = jnp.where((row // (D // H)) == col, f32(1.0), f32(0.0))

    hs = []
    for r in range(R):
        g = (g0_ref[...], g1_ref[...])[r]
        cnt = jnp.maximum(cnt4[2 * r + 0, :] + cnt4[2 * r + 1, :], 1.0)
        wct = dot(nwt_ref[r], twt_ref[r])
        node_msg = dot(g, wct) * (1.0 / (3.0 * cnt))[:, None]
        u = jnp.where(node_msg > 0, node_msg, jnp.exp(node_msg) - 1.0)
        a = _sigmoid(gx + dot(u, gwt[D:, :]))
        hs.append(jnp.tanh(u) * a + x_res * (1.0 - a))

    qkv = [dot(h, aiwt_ref[...]) + aib for h in hs]
    q = [z[:, 0:D] for z in qkv]
    k = [z[:, D:2 * D] for z in qkv]
    v = [z[:, 2 * D:3 * D] for z in qkv]

    scale = 1.0 / jnp.sqrt(f32(D // H))
    ts = []
    for r in range(R):
        s0 = dot(q[r] * k[0], P) * scale
        s1 = dot(q[r] * k[1], P) * scale
        mx = jnp.maximum(s0, s1)
        e0 = jnp.exp(s0 - mx)
        e1 = jnp.exp(s1 - mx)
        inv = 1.0 / (e0 + e1)
        a0 = e0 * inv
        a1 = e1 * inv
        a0b = a0[:, 0:1] * m0 + a0[:, 1:2] * m1
        a1b = a1[:, 0:1] * m0 + a1[:, 1:2] * m1
        o = a0b * v[0] + a1b * v[1]
        attn_out = dot(o, aowt_ref[...]) + aob
        t = _layernorm(hs[r] + attn_out, n1w, n1b)
        ff = dot(_gelu(dot(t, l1wt_ref[...]) + l1b), l2wt_ref[...]) + l2b
        ts.append(_layernorm(t + ff, n2w, n2b))

    out_ref[...] = (ts[0] + ts[1]) * 0.5


def _dense_call(x_pad, g, cnt4, nwt, twt, rwt, gwt, gate_b, aiwt, attn_in_b,
                aowt, attn_out_b, l1wt, lin1_b, l2wt, lin2_b,
                n1_w, n1_b, n2_w, n2_b):
    full = lambda shape: pl.BlockSpec(shape, lambda i: (0,) * len(shape))
    blk = lambda: pl.BlockSpec((BN, D), lambda i: (i, 0))
    return pl.pallas_call(
        _dense_body,
        grid=(GRID_N,),
        in_specs=[
            blk(),
            blk(), blk(),
            pl.BlockSpec((4, BN), lambda i: (0, i)),
            full((R, D, D)), full((R, D, D)),
            full((D, D)),
            full((2 * D, D)),
            full((D,)),
            full((D, 3 * D)), full((3 * D,)),
            full((D, D)), full((D,)),
            full((D, DFF)), full((DFF,)),
            full((DFF, D)), full((D,)),
            full((D,)), full((D,)),
            full((D,)), full((D,)),
        ],
        out_specs=blk(),
        out_shape=jax.ShapeDtypeStruct((N_PADC, D), jnp.float32),
    )(x_pad, g[0], g[1], cnt4, nwt, twt, rwt, gwt, gate_b, aiwt, attn_in_b,
      aowt, attn_out_b, l1wt, lin1_b, l2wt, lin2_b, n1_w, n1_b, n2_w, n2_b)


def kernel(x, edge_index_list, triangles_list, node_w, tri_w, res_w, gate_w,
           gate_b, attn_in_w, attn_in_b, attn_out_w, attn_out_b, lin1_w,
           lin1_b, lin2_w, lin2_b, n1_w, n1_b, n2_w, n2_b):
    del edge_index_list
    tri_pad = jnp.pad(triangles_list, ((0, 0), (0, 0), (0, T_PAD - T)),
                      constant_values=SENT)
    tri_flat = tri_pad.reshape(-1)
    m, cnt_part = _gather_sums(x, tri_flat)
    g = _scatter_sums(m, tri_flat)

    x_pad = jnp.pad(x, ((0, N_PADC - N), (0, 0)))
    cnt4 = cnt_part.reshape(NC * R, N_PADC)
    cnt4 = jnp.stack([cnt4[0], cnt4[2], cnt4[1], cnt4[3]])

    out = _dense_call(
        x_pad, g, cnt4,
        jnp.transpose(node_w, (0, 2, 1)), jnp.transpose(tri_w, (0, 2, 1)),
        res_w.T, gate_w.T, gate_b, attn_in_w.T, attn_in_b, attn_out_w.T,
        attn_out_b, lin1_w.T, lin1_b, lin2_w.T, lin2_b,
        n1_w, n1_b, n2_w, n2_b)
    return out[:N]
